# baseline (speedup 1.0000x reference)
import functools

import jax
import jax.numpy as jnp
from jax import lax
from jax.experimental import pallas as pl
from jax.experimental.pallas import tpu as pltpu
from jax.experimental.pallas import tpu_sc as plsc

B = 256
SC_CHUNK = 512
SC_G = 128
NUM_SC_WORKERS = 32


def _round_up(a, m):
    return (a + m - 1) // m * m


def _sc_gather(table, idx):
    T = idx.shape[0]
    D = table.shape[1]
    n_chunks = T // (NUM_SC_WORKERS * SC_CHUNK)
    assert n_chunks * NUM_SC_WORKERS * SC_CHUNK == T
    mesh = plsc.VectorSubcoreMesh(core_axis_name="c", subcore_axis_name="s")

    @functools.partial(
        pl.kernel,
        mesh=mesh,
        out_type=jax.ShapeDtypeStruct((T, D), table.dtype),
        scratch_types=[
            pltpu.VMEM((SC_CHUNK,), jnp.int32),
            pltpu.VMEM((SC_CHUNK, D), table.dtype),
            pltpu.SemaphoreType.DMA,
        ],
    )
    def k(table_hbm, idx_hbm, out_hbm, idx_v, rows_v, sem):
        wid = lax.axis_index("s") * 2 + lax.axis_index("c")

        @pl.loop(0, n_chunks)
        def _chunk(i):
            base = (i * NUM_SC_WORKERS + wid) * SC_CHUNK
            pltpu.sync_copy(idx_hbm.at[pl.ds(base, SC_CHUNK)], idx_v)

            @pl.loop(0, SC_CHUNK // SC_G)
            def _g(j):
                pltpu.async_copy(
                    table_hbm.at[idx_v.at[pl.ds(j * SC_G, SC_G)]],
                    rows_v.at[pl.ds(j * SC_G, SC_G)],
                    sem,
                ).wait()

            pltpu.sync_copy(rows_v, out_hbm.at[pl.ds(base, SC_CHUNK)])

    return k(table, idx)


def _sage_tc(gxbuf, feats, deg_pad, Wih, Whh, bsum, Wself, Wneigh, bcomb,
             gx_base, n_valid, Wc=None, bc=None):
    N_pad, D = feats.shape
    H = Wneigh.shape[0]
    final = Wc is not None
    nblocks = N_pad // B

    def body(*refs):
        if final:
            (deg_ref, feats_ref, wih_ref, whh_ref, bsum_ref, wself_ref,
             wneigh_ref, bcomb_ref, wc_ref, bc_ref, gx_ref, out_ref,
             logits_ref, h_ref, c_ref, slab_ref, sem_ref) = refs
        else:
            (deg_ref, feats_ref, wih_ref, whh_ref, bsum_ref, wself_ref,
             wneigh_ref, bcomb_ref, gx_ref, out_ref,
             h_ref, c_ref, slab_ref, sem_ref) = refs

        h_ref[...] = jnp.zeros((N_pad, H), jnp.float32)
        c_ref[...] = jnp.zeros((N_pad, H), jnp.float32)

        max_deg = deg_ref[0]

        def dread(c):
            return deg_ref[jnp.maximum(c - 1, 0)]

        def step(t, carry):
            off, cnt = carry
            def w_cond(cd):
                c, d = cd
                return (c > 0) & (d <= t)

            def w_body(cd):
                c, _ = cd
                c2 = c - 1
                return (c2, dread(c2))

            cnt, _ = lax.while_loop(w_cond, w_body, (cnt, dread(cnt)))
            nblk = (cnt + B - 1) // B

            def blk(jb, _):
                start = gx_base + off + jb * B
                cp = pltpu.make_async_copy(
                    gx_ref.at[pl.ds(start, B)], slab_ref, sem_ref)
                cp.start()
                cp.wait()
                hb = h_ref[pl.ds(jb * B, B), :]
                cb = c_ref[pl.ds(jb * B, B), :]
                gates = (
                    jnp.dot(slab_ref[...], wih_ref[...],
                            preferred_element_type=jnp.float32)
                    + jnp.dot(hb, whh_ref[...],
                              preferred_element_type=jnp.float32)
                    + bsum_ref[...]
                )
                gi = gates[:, 0:H]
                gf = gates[:, H:2 * H]
                gg = gates[:, 2 * H:3 * H]
                go = gates[:, 3 * H:4 * H]
                c_new = jax.nn.sigmoid(gf) * cb + \
                    jax.nn.sigmoid(gi) * jnp.tanh(gg)
                h_new = jax.nn.sigmoid(go) * jnp.tanh(c_new)
                ranks = jb * B + lax.broadcasted_iota(jnp.int32, (B, 1), 0)
                m = ranks < cnt
                h_ref[pl.ds(jb * B, B), :] = jnp.where(m, h_new, hb)
                c_ref[pl.ds(jb * B, B), :] = jnp.where(m, c_new, cb)
                return 0

            lax.fori_loop(0, nblk, blk, 0)
            return (off + cnt, cnt)

        lax.fori_loop(0, max_deg, step,
                      (jnp.int32(0), jnp.int32(N_pad)))

        def comb(jb, acc):
            fb = feats_ref[pl.ds(jb * B, B), :]
            hb = h_ref[pl.ds(jb * B, B), :]
            ob = (
                jnp.dot(fb, wself_ref[...],
                        preferred_element_type=jnp.float32)
                + jnp.dot(hb, wneigh_ref[...],
                          preferred_element_type=jnp.float32)
                + bcomb_ref[...]
            )
            ob = jnp.maximum(ob, 0.0)
            out_ref[pl.ds(jb * B, B), :] = ob
            if final:
                ranks = jb * B + lax.broadcasted_iota(jnp.int32, (B, 1), 0)
                obm = jnp.where(ranks < n_valid, ob, -jnp.inf)
                acc = jnp.maximum(acc, jnp.max(obm, axis=0, keepdims=True))
            return acc

        acc = lax.fori_loop(0, nblocks, comb,
                            jnp.full((1, H), -jnp.inf, jnp.float32))
        if final:
            logits_ref[...] = (
                jnp.dot(acc, wc_ref[...], preferred_element_type=jnp.float32)
                + bc_ref[...]
            )

    in_specs = [
        pl.BlockSpec(memory_space=pltpu.SMEM),
        pl.BlockSpec(memory_space=pltpu.VMEM),
        pl.BlockSpec(memory_space=pltpu.VMEM),
        pl.BlockSpec(memory_space=pltpu.VMEM),
        pl.BlockSpec(memory_space=pltpu.VMEM),
        pl.BlockSpec(memory_space=pltpu.VMEM),
        pl.BlockSpec(memory_space=pltpu.VMEM),
        pl.BlockSpec(memory_space=pltpu.VMEM),
    ]
    args = [deg_pad, feats, Wih, Whh, bsum, Wself, Wneigh, bcomb]
    if final:
        in_specs += [pl.BlockSpec(memory_space=pltpu.VMEM),
                     pl.BlockSpec(memory_space=pltpu.VMEM)]
        args += [Wc, bc]
    in_specs += [pl.BlockSpec(memory_space=pl.ANY)]
    args += [gxbuf]

    out_shape = [jax.ShapeDtypeStruct((N_pad, H), jnp.float32)]
    out_specs = [pl.BlockSpec(memory_space=pltpu.VMEM)]
    if final:
        out_shape.append(jax.ShapeDtypeStruct((1, Wc.shape[1]), jnp.float32))
        out_specs.append(pl.BlockSpec(memory_space=pltpu.VMEM))

    res = pl.pallas_call(
        body,
        grid=(),
        in_specs=in_specs,
        out_specs=out_specs,
        out_shape=out_shape,
        scratch_shapes=[
            pltpu.VMEM((N_pad, H), jnp.float32),
            pltpu.VMEM((N_pad, H), jnp.float32),
            pltpu.VMEM((B, D), jnp.float32),
            pltpu.SemaphoreType.DMA,
        ],
    )(*args)
    return res


def kernel(x, edge_index, Wih1, Whh1, bih1, bhh1, Wself1, Wneigh1, b1,
           Wih2, Whh2, bih2, bhh2, Wself2, Wneigh2, b2, Wc, bc):
    N, D = x.shape
    E = edge_index.shape[1]
    H = Whh1.shape[0]

    N_pad = _round_up(N, B)
    sc_quant = NUM_SC_WORKERS * SC_CHUNK
    T1 = _round_up(N_pad + E + B, sc_quant)
    T2 = _round_up(E + B, sc_quant)

    src = edge_index[0]
    dst = edge_index[1]
    order = jnp.argsort(dst, stable=True)
    src_s = src[order].astype(jnp.int32)
    dst_s = dst[order].astype(jnp.int32)
    deg = jnp.bincount(dst, length=N).astype(jnp.int32)
    starts = (jnp.cumsum(deg) - deg).astype(jnp.int32)
    t_k = jnp.arange(E, dtype=jnp.int32) - starts[dst_s]

    perm = jnp.argsort(-deg, stable=True).astype(jnp.int32)
    deg_sorted = deg[perm]
    rank = jnp.zeros((N,), jnp.int32).at[perm].set(
        jnp.arange(N, dtype=jnp.int32))

    hist = jnp.bincount(deg, length=E + 1)
    cnt_arr = (N - jnp.cumsum(hist)).astype(jnp.int32)
    off = jnp.concatenate([jnp.zeros((1,), jnp.int32),
                           jnp.cumsum(cnt_arr).astype(jnp.int32)])
    slot = off[t_k] + rank[dst_s]

    eidx1 = jnp.zeros((T1 - N_pad,), jnp.int32).at[slot].set(src_s)
    eidx2 = jnp.zeros((T2,), jnp.int32).at[slot].set(rank[src_s])
    perm_pad = jnp.zeros((N_pad,), jnp.int32).at[:N].set(perm)
    idx1 = jnp.concatenate([perm_pad, eidx1])

    deg_pad = jnp.zeros((N_pad,), jnp.int32).at[:N].set(deg_sorted)

    bsum1 = (bih1 + bhh1).reshape(1, 4 * H)
    bsum2 = (bih2 + bhh2).reshape(1, 4 * H)

    g1 = _sc_gather(x, idx1)
    xs = g1[:N_pad]
    (out1,) = _sage_tc(g1, xs, deg_pad, Wih1, Whh1, bsum1, Wself1, Wneigh1,
                       b1.reshape(1, H), gx_base=N_pad, n_valid=N)

    g2 = _sc_gather(out1, eidx2)
    out2, logits = _sage_tc(g2, out1, deg_pad, Wih2, Whh2, bsum2, Wself2,
                            Wneigh2, b2.reshape(1, H), gx_base=0, n_valid=N,
                            Wc=Wc, bc=bc.reshape(1, Wc.shape[1]))
    del out2
    return logits


# speedup vs baseline: 2.2176x; 2.2176x over previous
import functools

import jax
import jax.numpy as jnp
from jax import lax
from jax.experimental import pallas as pl
from jax.experimental.pallas import tpu as pltpu
from jax.experimental.pallas import tpu_sc as plsc

B = 256
SC_CHUNK = 512
SC_G = 128
NUM_SC_WORKERS = 32


def _round_up(a, m):
    return (a + m - 1) // m * m


def _sc_gather(table, idx, chunk=SC_CHUNK, g=SC_G):
    T = idx.shape[0]
    D = table.shape[1]
    n_chunks = T // (NUM_SC_WORKERS * chunk)
    assert n_chunks * NUM_SC_WORKERS * chunk == T and chunk % g == 0
    mesh = plsc.VectorSubcoreMesh(core_axis_name="c", subcore_axis_name="s")

    @functools.partial(
        pl.kernel,
        mesh=mesh,
        out_type=jax.ShapeDtypeStruct((T, D), table.dtype),
        scratch_types=[
            pltpu.VMEM((chunk,), jnp.int32),
            pltpu.VMEM((chunk, D), table.dtype),
            pltpu.SemaphoreType.DMA,
        ],
    )
    def k(table_hbm, idx_hbm, out_hbm, idx_v, rows_v, sem):
        wid = lax.axis_index("s") * 2 + lax.axis_index("c")

        @pl.loop(0, n_chunks)
        def _chunk(i):
            base = (i * NUM_SC_WORKERS + wid) * chunk
            pltpu.sync_copy(idx_hbm.at[pl.ds(base, chunk)], idx_v)

            @pl.loop(0, chunk // g)
            def _g(j):
                pltpu.async_copy(
                    table_hbm.at[idx_v.at[pl.ds(j * g, g)]],
                    rows_v.at[pl.ds(j * g, g)],
                    sem,
                ).wait()

            pltpu.sync_copy(rows_v, out_hbm.at[pl.ds(base, chunk)])

    return k(table, idx)


def _sage_tc(gxbuf, feats, deg_pad, Wih, Whh, bsum, Wself, Wneigh, bcomb,
             gx_base, n_valid, Wc=None, bc=None):
    N_pad, D = feats.shape
    H = Wneigh.shape[0]
    final = Wc is not None
    nblocks = N_pad // B

    def body(*refs):
        if final:
            (deg_ref, feats_ref, wih_ref, whh_ref, bsum_ref, wself_ref,
             wneigh_ref, bcomb_ref, wc_ref, bc_ref, gx_ref, out_ref,
             logits_ref, h_ref, c_ref, slab_ref, sem_ref) = refs
        else:
            (deg_ref, feats_ref, wih_ref, whh_ref, bsum_ref, wself_ref,
             wneigh_ref, bcomb_ref, gx_ref, out_ref,
             h_ref, c_ref, slab_ref, sem_ref) = refs

        h_ref[...] = jnp.zeros((N_pad, H), jnp.float32)
        c_ref[...] = jnp.zeros((N_pad, H), jnp.float32)

        max_deg = deg_ref[0]

        def dread(c):
            return deg_ref[jnp.maximum(c - 1, 0)]

        def step(t, carry):
            off, cnt = carry
            def w_cond(cd):
                c, d = cd
                return (c > 0) & (d <= t)

            def w_body(cd):
                c, _ = cd
                c2 = c - 1
                return (c2, dread(c2))

            cnt, _ = lax.while_loop(w_cond, w_body, (cnt, dread(cnt)))
            nblk = (cnt + B - 1) // B

            def blk(jb, _):
                start = gx_base + off + jb * B
                cp = pltpu.make_async_copy(
                    gx_ref.at[pl.ds(start, B)], slab_ref, sem_ref)
                cp.start()
                cp.wait()
                hb = h_ref[pl.ds(jb * B, B), :]
                cb = c_ref[pl.ds(jb * B, B), :]
                gates = (
                    jnp.dot(slab_ref[...], wih_ref[...],
                            preferred_element_type=jnp.float32)
                    + jnp.dot(hb, whh_ref[...],
                              preferred_element_type=jnp.float32)
                    + bsum_ref[...]
                )
                gi = gates[:, 0:H]
                gf = gates[:, H:2 * H]
                gg = gates[:, 2 * H:3 * H]
                go = gates[:, 3 * H:4 * H]
                c_new = jax.nn.sigmoid(gf) * cb + \
                    jax.nn.sigmoid(gi) * jnp.tanh(gg)
                h_new = jax.nn.sigmoid(go) * jnp.tanh(c_new)
                ranks = jb * B + lax.broadcasted_iota(jnp.int32, (B, 1), 0)
                m = ranks < cnt
                h_ref[pl.ds(jb * B, B), :] = jnp.where(m, h_new, hb)
                c_ref[pl.ds(jb * B, B), :] = jnp.where(m, c_new, cb)
                return 0

            lax.fori_loop(0, nblk, blk, 0)
            return (off + cnt, cnt)

        lax.fori_loop(0, max_deg, step,
                      (jnp.int32(0), jnp.int32(N_pad)))

        def comb(jb, acc):
            fb = feats_ref[pl.ds(jb * B, B), :]
            hb = h_ref[pl.ds(jb * B, B), :]
            ob = (
                jnp.dot(fb, wself_ref[...],
                        preferred_element_type=jnp.float32)
                + jnp.dot(hb, wneigh_ref[...],
                          preferred_element_type=jnp.float32)
                + bcomb_ref[...]
            )
            ob = jnp.maximum(ob, 0.0)
            out_ref[pl.ds(jb * B, B), :] = ob
            if final:
                ranks = jb * B + lax.broadcasted_iota(jnp.int32, (B, 1), 0)
                obm = jnp.where(ranks < n_valid, ob, -jnp.inf)
                acc = jnp.maximum(acc, jnp.max(obm, axis=0, keepdims=True))
            return acc

        acc = lax.fori_loop(0, nblocks, comb,
                            jnp.full((1, H), -jnp.inf, jnp.float32))
        if final:
            logits_ref[...] = (
                jnp.dot(acc, wc_ref[...], preferred_element_type=jnp.float32)
                + bc_ref[...]
            )

    in_specs = [
        pl.BlockSpec(memory_space=pltpu.SMEM),
        pl.BlockSpec(memory_space=pltpu.VMEM),
        pl.BlockSpec(memory_space=pltpu.VMEM),
        pl.BlockSpec(memory_space=pltpu.VMEM),
        pl.BlockSpec(memory_space=pltpu.VMEM),
        pl.BlockSpec(memory_space=pltpu.VMEM),
        pl.BlockSpec(memory_space=pltpu.VMEM),
        pl.BlockSpec(memory_space=pltpu.VMEM),
    ]
    args = [deg_pad, feats, Wih, Whh, bsum, Wself, Wneigh, bcomb]
    if final:
        in_specs += [pl.BlockSpec(memory_space=pltpu.VMEM),
                     pl.BlockSpec(memory_space=pltpu.VMEM)]
        args += [Wc, bc]
    in_specs += [pl.BlockSpec(memory_space=pl.ANY)]
    args += [gxbuf]

    out_shape = [jax.ShapeDtypeStruct((N_pad, H), jnp.float32)]
    out_specs = [pl.BlockSpec(memory_space=pltpu.VMEM)]
    if final:
        out_shape.append(jax.ShapeDtypeStruct((1, Wc.shape[1]), jnp.float32))
        out_specs.append(pl.BlockSpec(memory_space=pltpu.VMEM))

    res = pl.pallas_call(
        body,
        grid=(),
        in_specs=in_specs,
        out_specs=out_specs,
        out_shape=out_shape,
        scratch_shapes=[
            pltpu.VMEM((N_pad, H), jnp.float32),
            pltpu.VMEM((N_pad, H), jnp.float32),
            pltpu.VMEM((B, D), jnp.float32),
            pltpu.SemaphoreType.DMA,
        ],
    )(*args)
    return res


def kernel(x, edge_index, Wih1, Whh1, bih1, bhh1, Wself1, Wneigh1, b1,
           Wih2, Whh2, bih2, bhh2, Wself2, Wneigh2, b2, Wc, bc):
    N, D = x.shape
    E = edge_index.shape[1]
    H = Whh1.shape[0]

    N_pad = _round_up(N, B)
    sc_quant = NUM_SC_WORKERS * SC_CHUNK
    T1 = _round_up(N_pad + E + B, sc_quant)
    T2 = _round_up(E + B, sc_quant)

    src = edge_index[0].astype(jnp.int32)
    dst = edge_index[1].astype(jnp.int32)

    dst_s, src_s = lax.sort((dst, src), num_keys=1, is_stable=True)

    kidx = jnp.arange(E, dtype=jnp.int32)
    change = jnp.concatenate([jnp.ones((1,), jnp.bool_),
                              dst_s[1:] != dst_s[:-1]])
    seg_start = lax.cummax(jnp.where(change, kidx, 0))
    t_k = kidx - seg_start
    is_end = jnp.concatenate([dst_s[:-1] != dst_s[1:],
                              jnp.ones((1,), jnp.bool_)])
    seg_end = lax.cummin(jnp.where(is_end, kidx, E - 1), reverse=True)
    d_k = seg_end - seg_start + 1

    keyn_k = (E - d_k) * N + dst_s
    _, _, eidx = lax.sort((t_k, keyn_k, src_s), num_keys=2, is_stable=True)

    deg = jnp.bincount(dst, length=N).astype(jnp.int32)
    iota_n = jnp.arange(N, dtype=jnp.int32)
    key_nodes = (E - deg) * N + iota_n
    keys_sorted, perm = lax.sort((key_nodes, iota_n), num_keys=1)
    deg_sorted = E - (keys_sorted - perm) // N
    rank = jnp.zeros((N,), jnp.int32).at[perm].set(iota_n)

    eidx1 = jnp.zeros((T1 - N_pad,), jnp.int32).at[:E].set(eidx)
    eidx2 = jnp.zeros((T2,), jnp.int32).at[:E].set(eidx)
    perm_pad = jnp.zeros((N_pad,), jnp.int32).at[:N].set(perm)
    rank_pad = jnp.zeros((N_pad,), jnp.int32).at[:N].set(rank)
    idx1 = jnp.concatenate([perm_pad, eidx1])

    deg_pad = jnp.zeros((N_pad,), jnp.int32).at[:N].set(deg_sorted)

    bsum1 = (bih1 + bhh1).reshape(1, 4 * H)
    bsum2 = (bih2 + bhh2).reshape(1, 4 * H)

    g1 = _sc_gather(x, idx1)
    xs = g1[:N_pad]
    (out1,) = _sage_tc(g1, xs, deg_pad, Wih1, Whh1, bsum1, Wself1, Wneigh1,
                       b1.reshape(1, H), gx_base=N_pad, n_valid=N)

    out1_orig = _sc_gather(out1, rank_pad, chunk=N_pad // NUM_SC_WORKERS,
                           g=64)
    g2 = _sc_gather(out1_orig, eidx2)
    out2, logits = _sage_tc(g2, out1, deg_pad, Wih2, Whh2, bsum2, Wself2,
                            Wneigh2, b2.reshape(1, H), gx_base=0, n_valid=N,
                            Wc=Wc, bc=bc.reshape(1, Wc.shape[1]))
    del out2
    return logits


# speedup vs baseline: 2.9786x; 1.3432x over previous
import functools

import jax
import jax.numpy as jnp
from jax import lax
from jax.experimental import pallas as pl
from jax.experimental.pallas import tpu as pltpu
from jax.experimental.pallas import tpu_sc as plsc

B = 256
SC_CHUNK = 512
SC_G = 128
NUM_SC_WORKERS = 32


def _round_up(a, m):
    return (a + m - 1) // m * m


def _sc_gather(table, idx, chunk=SC_CHUNK, g=SC_G):
    T = idx.shape[0]
    D = table.shape[1]
    n_chunks = T // (NUM_SC_WORKERS * chunk)
    assert n_chunks * NUM_SC_WORKERS * chunk == T and chunk % g == 0
    mesh = plsc.VectorSubcoreMesh(core_axis_name="c", subcore_axis_name="s")

    @functools.partial(
        pl.kernel,
        mesh=mesh,
        out_type=jax.ShapeDtypeStruct((T, D), table.dtype),
        scratch_types=[
            pltpu.VMEM((chunk,), jnp.int32),
            pltpu.VMEM((chunk, D), table.dtype),
            pltpu.SemaphoreType.DMA,
        ],
    )
    def k(table_hbm, idx_hbm, out_hbm, idx_v, rows_v, sem):
        wid = lax.axis_index("s") * 2 + lax.axis_index("c")

        @pl.loop(0, n_chunks)
        def _chunk(i):
            base = (i * NUM_SC_WORKERS + wid) * chunk
            pltpu.sync_copy(idx_hbm.at[pl.ds(base, chunk)], idx_v)

            @pl.loop(0, chunk // g)
            def _g(j):
                pltpu.async_copy(
                    table_hbm.at[idx_v.at[pl.ds(j * g, g)]],
                    rows_v.at[pl.ds(j * g, g)],
                    sem,
                ).wait()

            pltpu.sync_copy(rows_v, out_hbm.at[pl.ds(base, chunk)])

    return k(table, idx)


def _sage_tc(gxbuf, feats, deg_pad, Wih, Whh, bsum, Wself, Wneigh, bcomb,
             gx_base, n_valid, Wc=None, bc=None):
    N_pad, D = feats.shape
    H = Wneigh.shape[0]
    final = Wc is not None
    nblocks = N_pad // B

    def body(*refs):
        if final:
            (deg_ref, feats_ref, wih_ref, whh_ref, bsum_ref, wself_ref,
             wneigh_ref, bcomb_ref, wc_ref, bc_ref, gx_ref, out_ref,
             logits_ref, h_ref, c_ref, slab_ref, sem_ref) = refs
        else:
            (deg_ref, feats_ref, wih_ref, whh_ref, bsum_ref, wself_ref,
             wneigh_ref, bcomb_ref, gx_ref, out_ref,
             h_ref, c_ref, slab_ref, sem_ref) = refs

        h_ref[...] = jnp.zeros((N_pad, H), jnp.float32)
        c_ref[...] = jnp.zeros((N_pad, H), jnp.float32)

        max_deg = deg_ref[0]

        def dread(c):
            return deg_ref[jnp.maximum(c - 1, 0)]

        def issue(buf, start):
            pltpu.make_async_copy(
                gx_ref.at[pl.ds(start, B)],
                slab_ref.at[buf], sem_ref.at[buf]).start()

        def wait(buf):
            pltpu.make_async_copy(
                gx_ref.at[pl.ds(gx_base, B)],
                slab_ref.at[buf], sem_ref.at[buf]).wait()

        issue(jnp.int32(0), jnp.int32(gx_base))

        def step(t, carry):
            off, cnt, i = carry
            def w_cond(cd):
                c, d = cd
                return (c > 0) & (d <= t)

            def w_body(cd):
                c, _ = cd
                c2 = c - 1
                return (c2, dread(c2))

            cnt, _ = lax.while_loop(w_cond, w_body, (cnt, dread(cnt)))
            nblk = (cnt + B - 1) // B

            def blk(jb, i_):
                nxt = jnp.where(jb + 1 < nblk, off + (jb + 1) * B, off + cnt)
                issue((i_ + 1) % 2, gx_base + nxt)
                buf = i_ % 2
                wait(buf)
                hb = h_ref[pl.ds(jb * B, B), :]
                cb = c_ref[pl.ds(jb * B, B), :]
                gates = (
                    jnp.dot(slab_ref[buf].astype(jnp.bfloat16), wih_ref[...],
                            preferred_element_type=jnp.float32)
                    + jnp.dot(hb.astype(jnp.bfloat16), whh_ref[...],
                              preferred_element_type=jnp.float32)
                    + bsum_ref[...]
                )
                gi = gates[:, 0:H]
                gf = gates[:, H:2 * H]
                gg = gates[:, 2 * H:3 * H]
                go = gates[:, 3 * H:4 * H]
                c_new = jax.nn.sigmoid(gf) * cb + \
                    jax.nn.sigmoid(gi) * jnp.tanh(gg)
                h_new = jax.nn.sigmoid(go) * jnp.tanh(c_new)
                ranks = jb * B + lax.broadcasted_iota(jnp.int32, (B, 1), 0)
                m = ranks < cnt
                h_ref[pl.ds(jb * B, B), :] = jnp.where(m, h_new, hb)
                c_ref[pl.ds(jb * B, B), :] = jnp.where(m, c_new, cb)
                return i_ + 1

            i = lax.fori_loop(0, nblk, blk, i)
            return (off + cnt, cnt, i)

        _, _, i_fin = lax.fori_loop(
            0, max_deg, step, (jnp.int32(0), jnp.int32(N_pad), jnp.int32(0)))
        wait(i_fin % 2)

        def comb(jb, acc):
            fb = feats_ref[pl.ds(jb * B, B), :]
            hb = h_ref[pl.ds(jb * B, B), :]
            ob = (
                jnp.dot(fb.astype(jnp.bfloat16), wself_ref[...],
                        preferred_element_type=jnp.float32)
                + jnp.dot(hb.astype(jnp.bfloat16), wneigh_ref[...],
                          preferred_element_type=jnp.float32)
                + bcomb_ref[...]
            )
            ob = jnp.maximum(ob, 0.0)
            out_ref[pl.ds(jb * B, B), :] = ob
            if final:
                ranks = jb * B + lax.broadcasted_iota(jnp.int32, (B, 1), 0)
                obm = jnp.where(ranks < n_valid, ob, -jnp.inf)
                acc = jnp.maximum(acc, jnp.max(obm, axis=0, keepdims=True))
            return acc

        acc = lax.fori_loop(0, nblocks, comb,
                            jnp.full((1, H), -jnp.inf, jnp.float32))
        if final:
            logits_ref[...] = (
                jnp.dot(acc, wc_ref[...], preferred_element_type=jnp.float32)
                + bc_ref[...]
            )

    in_specs = [
        pl.BlockSpec(memory_space=pltpu.SMEM),
        pl.BlockSpec(memory_space=pltpu.VMEM),
        pl.BlockSpec(memory_space=pltpu.VMEM),
        pl.BlockSpec(memory_space=pltpu.VMEM),
        pl.BlockSpec(memory_space=pltpu.VMEM),
        pl.BlockSpec(memory_space=pltpu.VMEM),
        pl.BlockSpec(memory_space=pltpu.VMEM),
        pl.BlockSpec(memory_space=pltpu.VMEM),
    ]
    args = [deg_pad, feats, Wih, Whh, bsum, Wself, Wneigh, bcomb]
    if final:
        in_specs += [pl.BlockSpec(memory_space=pltpu.VMEM),
                     pl.BlockSpec(memory_space=pltpu.VMEM)]
        args += [Wc, bc]
    in_specs += [pl.BlockSpec(memory_space=pl.ANY)]
    args += [gxbuf]

    out_shape = [jax.ShapeDtypeStruct((N_pad, H), jnp.float32)]
    out_specs = [pl.BlockSpec(memory_space=pltpu.VMEM)]
    if final:
        out_shape.append(jax.ShapeDtypeStruct((1, Wc.shape[1]), jnp.float32))
        out_specs.append(pl.BlockSpec(memory_space=pltpu.VMEM))

    res = pl.pallas_call(
        body,
        grid=(),
        in_specs=in_specs,
        out_specs=out_specs,
        out_shape=out_shape,
        scratch_shapes=[
            pltpu.VMEM((N_pad, H), jnp.float32),
            pltpu.VMEM((N_pad, H), jnp.float32),
            pltpu.VMEM((2, B, D), jnp.float32),
            pltpu.SemaphoreType.DMA((2,)),
        ],
    )(*args)
    return res


def kernel(x, edge_index, Wih1, Whh1, bih1, bhh1, Wself1, Wneigh1, b1,
           Wih2, Whh2, bih2, bhh2, Wself2, Wneigh2, b2, Wc, bc):
    N, D = x.shape
    E = edge_index.shape[1]
    H = Whh1.shape[0]

    N_pad = _round_up(N, B)
    sc_quant = NUM_SC_WORKERS * SC_CHUNK
    T1 = _round_up(N_pad + E + B, sc_quant)
    T2 = _round_up(E + B, sc_quant)

    src = edge_index[0].astype(jnp.int32)
    dst = edge_index[1].astype(jnp.int32)

    dst_s, src_s = lax.sort((dst, src), num_keys=1, is_stable=True)

    kidx = jnp.arange(E, dtype=jnp.int32)
    change = jnp.concatenate([jnp.ones((1,), jnp.bool_),
                              dst_s[1:] != dst_s[:-1]])
    seg_start = lax.cummax(jnp.where(change, kidx, 0))
    t_k = kidx - seg_start
    is_end = jnp.concatenate([dst_s[:-1] != dst_s[1:],
                              jnp.ones((1,), jnp.bool_)])
    seg_end = lax.cummin(jnp.where(is_end, kidx, E - 1), reverse=True)
    d_k = seg_end - seg_start + 1

    keyn_k = (E - d_k) * N + dst_s
    _, _, eidx = lax.sort((t_k, keyn_k, src_s), num_keys=2, is_stable=True)

    deg = jnp.bincount(dst, length=N).astype(jnp.int32)
    iota_n = jnp.arange(N, dtype=jnp.int32)
    key_nodes = (E - deg) * N + iota_n
    keys_sorted, perm = lax.sort((key_nodes, iota_n), num_keys=1)
    deg_sorted = E - (keys_sorted - perm) // N
    rank = jnp.zeros((N,), jnp.int32).at[perm].set(iota_n)

    eidx1 = jnp.zeros((T1 - N_pad,), jnp.int32).at[:E].set(eidx)
    eidx2 = jnp.zeros((T2,), jnp.int32).at[:E].set(eidx)
    perm_pad = jnp.zeros((N_pad,), jnp.int32).at[:N].set(perm)
    rank_pad = jnp.zeros((N_pad,), jnp.int32).at[:N].set(rank)
    idx1 = jnp.concatenate([perm_pad, eidx1])

    deg_pad = jnp.zeros((N_pad,), jnp.int32).at[:N].set(deg_sorted)

    bsum1 = (bih1 + bhh1).reshape(1, 4 * H)
    bsum2 = (bih2 + bhh2).reshape(1, 4 * H)

    bf = jnp.bfloat16

    g1 = _sc_gather(x, idx1)
    xs = g1[:N_pad]
    (out1,) = _sage_tc(g1, xs, deg_pad, Wih1.astype(bf), Whh1.astype(bf),
                       bsum1, Wself1.astype(bf), Wneigh1.astype(bf),
                       b1.reshape(1, H), gx_base=N_pad, n_valid=N)

    out1_orig = _sc_gather(out1, rank_pad, chunk=N_pad // NUM_SC_WORKERS,
                           g=64)
    g2 = _sc_gather(out1_orig, eidx2)
    out2, logits = _sage_tc(g2, out1, deg_pad, Wih2.astype(bf),
                            Whh2.astype(bf), bsum2, Wself2.astype(bf),
                            Wneigh2.astype(bf), b2.reshape(1, H), gx_base=0,
                            n_valid=N, Wc=Wc,
                            bc=bc.reshape(1, Wc.shape[1]))
    del out2
    return logits
